# baseline (speedup 1.0000x reference)
import functools

import jax
import jax.numpy as jnp
from jax import lax
from jax.experimental import pallas as pl
from jax.experimental.pallas import tpu as pltpu
from jax.experimental.pallas import tpu_sc as plsc

N = 10000
E = 320000
SUB = 100
NB = 320
NSLOT = 3
SUB_F = 80
NB_F = 400
NSLOT_F = 4
RPT = 632
RPT_LAST = N - 15 * RPT
SPLIT = 15 * RPT


def _make_sc_agg(split_edges):
    mesh = plsc.VectorSubcoreMesh(core_axis_name="c", subcore_axis_name="s")
    if split_edges:
        out_type = jax.ShapeDtypeStruct((2, N, 128), jnp.float32)
        nslot, sub, nbrows = NSLOT, SUB, NB
        nbody = nbrows // 32
    else:
        out_type = jax.ShapeDtypeStruct((N, 256), jnp.float32)
        nslot, sub, nbrows = NSLOT_F, SUB_F, NB_F
        nbody = nbrows // 16

    def body(h_hbm, src_hbm, dst_hbm, zeros_hbm, out_hbm,
             srcA, srcB, dstA, dstB, rows_v, agg_sh,
             sem_g, sem_s, sem_iA, sem_iB):
        c = lax.axis_index("c")
        s = lax.axis_index("s")
        if split_edges:
            m0 = (s * 2 + c) * nbody
            def src_slice(m, half):
                return src_hbm.at[m, half]
        else:
            m0 = s * nbody
            def src_slice(m, half):
                return src_hbm.at[c, m, half]

        def init(r0, nrows):
            pltpu.sync_copy(zeros_hbm.at[pl.ds(r0, nrows)],
                            agg_sh.at[pl.ds(r0, nrows)])

        def copy_out(r0, nrows):
            if split_edges:
                pltpu.sync_copy(agg_sh.at[pl.ds(r0, nrows)],
                                out_hbm.at[c, pl.ds(r0, nrows)])
            else:
                pltpu.sync_copy(agg_sh.at[pl.ds(r0, nrows)],
                                out_hbm.at[pl.ds(r0, nrows),
                                           pl.ds(c * 128, 128)])

        @pl.when(s < 15)
        def _():
            init(s * RPT, RPT)
        @pl.when(s == 15)
        def _():
            init(SPLIT, RPT_LAST)
        plsc.subcore_barrier()

        pltpu.sync_copy(src_slice(m0, 0), srcA)
        pltpu.sync_copy(dst_hbm.at[m0, 0], dstA)

        def drain_scatter(b):
            pltpu.make_async_copy(rows_v.at[b], agg_sh.at[dstA.at[0]],
                                  sem_s.at[b]).wait()

        def drain_idx(sem, src_ref, dst_ref):
            pltpu.make_async_copy(src_slice(m0, 0), src_ref, sem).wait()
            pltpu.make_async_copy(dst_hbm.at[m0, 0], dst_ref, sem).wait()

        @pl.loop(0, nbody)
        def _body(t):
            m = m0 + t
            @pl.when(t > 0)
            def _():
                for jj in range(10 - nslot, 10):
                    drain_scatter(jj % nslot)
                drain_idx(sem_iA, srcA, dstA)

            gather_descs = {}

            def fire_gather(j):
                b = j % nslot
                si = (srcA if j < 5 else srcB).at[j % 5]
                gather_descs[j] = pltpu.async_copy(
                    h_hbm.at[si], rows_v.at[b], sem_g.at[b])

            def fire_scatter(j):
                b = j % nslot
                di = (dstA if j < 5 else dstB).at[j % 5]
                gather_descs[j].wait()
                pltpu.async_copy(rows_v.at[b], agg_sh.at[di],
                                 sem_s.at[b], add=True)

            for j in range(10):
                if nslot <= j:
                    drain_scatter(j % nslot)
                fire_gather(j)
                if j == 2:
                    pltpu.async_copy(src_slice(m, 1), srcB, sem_iB)
                    pltpu.async_copy(dst_hbm.at[m, 1], dstB, sem_iB)
                if j == 5:
                    pltpu.make_async_copy(src_slice(m, 1), srcB, sem_iB).wait()
                    pltpu.make_async_copy(dst_hbm.at[m, 1], dstB, sem_iB).wait()
                if j == 4 + nslot:
                    mn = jnp.minimum(m + 1, nbrows - 1)
                    pltpu.async_copy(src_slice(mn, 0), srcA, sem_iA)
                    pltpu.async_copy(dst_hbm.at[mn, 0], dstA, sem_iA)
                if j >= 1:
                    fire_scatter(j - 1)
            fire_scatter(9)

        for jj in range(10 - nslot, 10):
            drain_scatter(jj % nslot)
        drain_idx(sem_iA, srcA, dstA)

        plsc.subcore_barrier()

        @pl.when(s < 15)
        def _():
            copy_out(s * RPT, RPT)
        @pl.when(s == 15)
        def _():
            copy_out(SPLIT, RPT_LAST)

    return pl.kernel(
        body,
        out_type=out_type,
        mesh=mesh,
        scratch_types=[
            pltpu.VMEM((5, sub), jnp.int32),
            pltpu.VMEM((5, sub), jnp.int32),
            pltpu.VMEM((5, sub), jnp.int32),
            pltpu.VMEM((5, sub), jnp.int32),
            pltpu.VMEM((nslot, sub, 128), jnp.float32),
            pltpu.VMEM_SHARED((N, 128), jnp.float32),
            pltpu.SemaphoreType.DMA((nslot,)),
            pltpu.SemaphoreType.DMA((nslot,)),
            pltpu.SemaphoreType.DMA,
            pltpu.SemaphoreType.DMA,
        ],
    )


_SC_AGG128 = _make_sc_agg(split_edges=False)
_SC_AGG_L0 = _make_sc_agg(split_edges=True)

BR = 1000


def _mlp_block(hin, W1_ref, b1_ref, g_ref, be_ref, W2_ref, b2_ref):
    z = jnp.dot(hin, W1_ref[...], preferred_element_type=jnp.float32) + b1_ref[...]
    mu = jnp.mean(z, axis=-1, keepdims=True)
    zc = z - mu
    var = jnp.mean(zc * zc, axis=-1, keepdims=True)
    zn = zc * lax.rsqrt(var + 1e-5) * g_ref[...] + be_ref[...]
    za = jnp.maximum(zn, 0.0)
    return jnp.dot(za, W2_ref[...], preferred_element_type=jnp.float32) + b2_ref[...]


def _wspecs(din):
    return [
        pl.BlockSpec((1, 1), lambda i: (0, 0)),
        pl.BlockSpec((din, 256), lambda i: (0, 0)),
        pl.BlockSpec((1, 256), lambda i: (0, 0)),
        pl.BlockSpec((1, 256), lambda i: (0, 0)),
        pl.BlockSpec((1, 256), lambda i: (0, 0)),
        pl.BlockSpec((256, 256), lambda i: (0, 0)),
        pl.BlockSpec((1, 256), lambda i: (0, 0)),
    ]


def _tc_layer0(x, agg, eps, W1, b1, g, be, W2, b2):
    def body(eps_ref, W1_ref, b1_ref, g_ref, be_ref, W2_ref, b2_ref,
             x_ref, agg_ref, out_ref):
        hin = (1.0 + eps_ref[0, 0]) * x_ref[...] + (agg_ref[0] + agg_ref[1])
        o = _mlp_block(hin, W1_ref, b1_ref, g_ref, be_ref, W2_ref, b2_ref)
        h1 = jnp.maximum(o, 0.0)
        out_ref[0] = h1[:, :128]
        out_ref[1] = h1[:, 128:]

    return pl.pallas_call(
        body,
        grid=(N // BR,),
        in_specs=_wspecs(128) + [
            pl.BlockSpec((BR, 128), lambda i: (i, 0)),
            pl.BlockSpec((2, BR, 128), lambda i: (0, i, 0)),
        ],
        out_specs=pl.BlockSpec((2, BR, 128), lambda i: (0, i, 0)),
        out_shape=jax.ShapeDtypeStruct((2, N, 128), jnp.float32),
    )(eps.reshape(1, 1), W1, b1.reshape(1, 256), g.reshape(1, 256),
      be.reshape(1, 256), W2, b2.reshape(1, 256), x, agg)


def _tc_layer_mid(hh, agg, eps, W1, b1, g, be, W2, b2):
    def body(eps_ref, W1_ref, b1_ref, g_ref, be_ref, W2_ref, b2_ref,
             hh_ref, agg_ref, out_ref):
        h = jnp.concatenate([hh_ref[0], hh_ref[1]], axis=-1)
        hin = (1.0 + eps_ref[0, 0]) * h + agg_ref[...]
        o = _mlp_block(hin, W1_ref, b1_ref, g_ref, be_ref, W2_ref, b2_ref)
        h2 = h + jnp.maximum(o, 0.0)
        out_ref[0] = h2[:, :128]
        out_ref[1] = h2[:, 128:]

    return pl.pallas_call(
        body,
        grid=(N // BR,),
        in_specs=_wspecs(256) + [
            pl.BlockSpec((2, BR, 128), lambda i: (0, i, 0)),
            pl.BlockSpec((BR, 256), lambda i: (i, 0)),
        ],
        out_specs=pl.BlockSpec((2, BR, 128), lambda i: (0, i, 0)),
        out_shape=jax.ShapeDtypeStruct((2, N, 128), jnp.float32),
    )(eps.reshape(1, 1), W1, b1.reshape(1, 256), g.reshape(1, 256),
      be.reshape(1, 256), W2, b2.reshape(1, 256), hh, agg)


def _tc_layer_last(hh, agg, eps, W1, b1, g, be, W2, b2, Wo_pad, bo_pad):
    def body(eps_ref, W1_ref, b1_ref, g_ref, be_ref, W2_ref, b2_ref,
             Wo_ref, bo_ref, hh_ref, agg_ref, out_ref):
        h = jnp.concatenate([hh_ref[0], hh_ref[1]], axis=-1)
        hin = (1.0 + eps_ref[0, 0]) * h + agg_ref[...]
        o = _mlp_block(hin, W1_ref, b1_ref, g_ref, be_ref, W2_ref, b2_ref)
        h3 = h + jnp.maximum(o, 0.0)
        out_ref[...] = (jnp.dot(h3, Wo_ref[...], preferred_element_type=jnp.float32)
                        + bo_ref[...])

    return pl.pallas_call(
        body,
        grid=(N // BR,),
        in_specs=_wspecs(256) + [
            pl.BlockSpec((256, 128), lambda i: (0, 0)),
            pl.BlockSpec((1, 128), lambda i: (0, 0)),
            pl.BlockSpec((2, BR, 128), lambda i: (0, i, 0)),
            pl.BlockSpec((BR, 256), lambda i: (i, 0)),
        ],
        out_specs=pl.BlockSpec((BR, 128), lambda i: (i, 0)),
        out_shape=jax.ShapeDtypeStruct((N, 128), jnp.float32),
    )(eps.reshape(1, 1), W1, b1.reshape(1, 256), g.reshape(1, 256),
      be.reshape(1, 256), W2, b2.reshape(1, 256), Wo_pad, bo_pad, hh, agg)


def kernel(x, edge_index,
           W1_0, b1_0, g_0, be_0, W2_0, b2_0, eps_0,
           W1_1, b1_1, g_1, be_1, W2_1, b2_1, eps_1,
           W1_2, b1_2, g_2, be_2, W2_2, b2_2, eps_2,
           W_out, b_out):
    src = edge_index[0].astype(jnp.int32)
    dst = edge_index[1].astype(jnp.int32)
    srcs = jnp.stack([src, src + N]).reshape(2, NB_F, 2, 5, SUB_F)
    dst2 = dst.reshape(NB_F, 2, 5, SUB_F)
    src0 = src.reshape(NB, 2, 5, SUB)
    dst0 = dst.reshape(NB, 2, 5, SUB)
    z128 = jnp.zeros((N, 128), jnp.float32)

    agg0 = _SC_AGG_L0(x, src0, dst0, z128)
    h1h = _tc_layer0(x, agg0, eps_0, W1_0, b1_0, g_0, be_0, W2_0, b2_0)
    agg1 = _SC_AGG128(h1h.reshape(2 * N, 128), srcs, dst2, z128)
    h2h = _tc_layer_mid(h1h, agg1, eps_1, W1_1, b1_1, g_1, be_1, W2_1, b2_1)
    agg2 = _SC_AGG128(h2h.reshape(2 * N, 128), srcs, dst2, z128)
    Wo_pad = jnp.pad(W_out, ((0, 0), (0, 126)))
    bo_pad = jnp.pad(b_out, (0, 126)).reshape(1, 128)
    outp = _tc_layer_last(h2h, agg2, eps_2, W1_2, b1_2, g_2, be_2, W2_2, b2_2,
                          Wo_pad, bo_pad)
    return outp[:, :2]


# speedup vs baseline: 1.0523x; 1.0523x over previous
import functools

import jax
import jax.numpy as jnp
from jax import lax
from jax.experimental import pallas as pl
from jax.experimental.pallas import tpu as pltpu
from jax.experimental.pallas import tpu_sc as plsc

N = 10000
E = 320000
SUB = 100
NB = 320
NSLOT = 3
SUB_F = 125
NB_F = 256
NSLOT_F = 2
RPT = 632
RPT_LAST = N - 15 * RPT
SPLIT = 15 * RPT


def _make_sc_agg(split_edges):
    mesh = plsc.VectorSubcoreMesh(core_axis_name="c", subcore_axis_name="s")
    if split_edges:
        out_type = jax.ShapeDtypeStruct((2, N, 128), jnp.float32)
        nslot, sub, nbrows = NSLOT_F, SUB_F, NB_F
        nbody = nbrows // 32
    else:
        out_type = jax.ShapeDtypeStruct((N, 256), jnp.float32)
        nslot, sub, nbrows = NSLOT_F, SUB_F, NB_F
        nbody = nbrows // 16

    def body(h_hbm, src_hbm, dst_hbm, zeros_hbm, out_hbm,
             srcA, srcB, dstA, dstB, rows_v, agg_sh,
             sem_g, sem_s, sem_iA, sem_iB):
        c = lax.axis_index("c")
        s = lax.axis_index("s")
        if split_edges:
            m0 = (s * 2 + c) * nbody
            def src_slice(m, half):
                return src_hbm.at[m, half]
        else:
            m0 = s * nbody
            def src_slice(m, half):
                return src_hbm.at[c, m, half]

        def init(r0, nrows):
            pltpu.sync_copy(zeros_hbm.at[pl.ds(r0, nrows)],
                            agg_sh.at[pl.ds(r0, nrows)])

        def copy_out(r0, nrows):
            if split_edges:
                pltpu.sync_copy(agg_sh.at[pl.ds(r0, nrows)],
                                out_hbm.at[c, pl.ds(r0, nrows)])
            else:
                pltpu.sync_copy(agg_sh.at[pl.ds(r0, nrows)],
                                out_hbm.at[pl.ds(r0, nrows),
                                           pl.ds(c * 128, 128)])

        @pl.when(s < 15)
        def _():
            init(s * RPT, RPT)
        @pl.when(s == 15)
        def _():
            init(SPLIT, RPT_LAST)
        plsc.subcore_barrier()

        pltpu.sync_copy(src_slice(m0, 0), srcA)
        pltpu.sync_copy(dst_hbm.at[m0, 0], dstA)

        def drain_scatter(b):
            pltpu.make_async_copy(rows_v.at[b], agg_sh.at[dstA.at[0]],
                                  sem_s.at[b]).wait()

        def drain_idx(sem, src_ref, dst_ref):
            pltpu.make_async_copy(src_slice(m0, 0), src_ref, sem).wait()
            pltpu.make_async_copy(dst_hbm.at[m0, 0], dst_ref, sem).wait()

        @pl.loop(0, nbody)
        def _body(t):
            m = m0 + t
            @pl.when(t > 0)
            def _():
                for jj in range(10 - nslot, 10):
                    drain_scatter(jj % nslot)
                drain_idx(sem_iA, srcA, dstA)

            gather_descs = {}

            def fire_gather(j):
                b = j % nslot
                si = (srcA if j < 5 else srcB).at[j % 5]
                gather_descs[j] = pltpu.async_copy(
                    h_hbm.at[si], rows_v.at[b], sem_g.at[b])

            def fire_scatter(j):
                b = j % nslot
                di = (dstA if j < 5 else dstB).at[j % 5]
                gather_descs[j].wait()
                pltpu.async_copy(rows_v.at[b], agg_sh.at[di],
                                 sem_s.at[b], add=True)

            for j in range(10):
                if nslot <= j:
                    drain_scatter(j % nslot)
                fire_gather(j)
                if j == 2:
                    pltpu.async_copy(src_slice(m, 1), srcB, sem_iB)
                    pltpu.async_copy(dst_hbm.at[m, 1], dstB, sem_iB)
                if j == 5:
                    pltpu.make_async_copy(src_slice(m, 1), srcB, sem_iB).wait()
                    pltpu.make_async_copy(dst_hbm.at[m, 1], dstB, sem_iB).wait()
                if j == 4 + nslot:
                    mn = jnp.minimum(m + 1, nbrows - 1)
                    pltpu.async_copy(src_slice(mn, 0), srcA, sem_iA)
                    pltpu.async_copy(dst_hbm.at[mn, 0], dstA, sem_iA)
                if j >= 1:
                    fire_scatter(j - 1)
            fire_scatter(9)

        for jj in range(10 - nslot, 10):
            drain_scatter(jj % nslot)
        drain_idx(sem_iA, srcA, dstA)

        plsc.subcore_barrier()

        @pl.when(s < 15)
        def _():
            copy_out(s * RPT, RPT)
        @pl.when(s == 15)
        def _():
            copy_out(SPLIT, RPT_LAST)

    return pl.kernel(
        body,
        out_type=out_type,
        mesh=mesh,
        scratch_types=[
            pltpu.VMEM((5, sub), jnp.int32),
            pltpu.VMEM((5, sub), jnp.int32),
            pltpu.VMEM((5, sub), jnp.int32),
            pltpu.VMEM((5, sub), jnp.int32),
            pltpu.VMEM((nslot, sub, 128), jnp.float32),
            pltpu.VMEM_SHARED((N, 128), jnp.float32),
            pltpu.SemaphoreType.DMA((nslot,)),
            pltpu.SemaphoreType.DMA((nslot,)),
            pltpu.SemaphoreType.DMA,
            pltpu.SemaphoreType.DMA,
        ],
    )


_SC_AGG128 = _make_sc_agg(split_edges=False)
_SC_AGG_L0 = _make_sc_agg(split_edges=True)

BR = 1000


def _mlp_block(hin, W1_ref, b1_ref, g_ref, be_ref, W2_ref, b2_ref):
    z = jnp.dot(hin, W1_ref[...], preferred_element_type=jnp.float32) + b1_ref[...]
    mu = jnp.mean(z, axis=-1, keepdims=True)
    zc = z - mu
    var = jnp.mean(zc * zc, axis=-1, keepdims=True)
    zn = zc * lax.rsqrt(var + 1e-5) * g_ref[...] + be_ref[...]
    za = jnp.maximum(zn, 0.0)
    return jnp.dot(za, W2_ref[...], preferred_element_type=jnp.float32) + b2_ref[...]


def _wspecs(din):
    return [
        pl.BlockSpec((1, 1), lambda i: (0, 0)),
        pl.BlockSpec((din, 256), lambda i: (0, 0)),
        pl.BlockSpec((1, 256), lambda i: (0, 0)),
        pl.BlockSpec((1, 256), lambda i: (0, 0)),
        pl.BlockSpec((1, 256), lambda i: (0, 0)),
        pl.BlockSpec((256, 256), lambda i: (0, 0)),
        pl.BlockSpec((1, 256), lambda i: (0, 0)),
    ]


def _tc_layer0(x, agg, eps, W1, b1, g, be, W2, b2):
    def body(eps_ref, W1_ref, b1_ref, g_ref, be_ref, W2_ref, b2_ref,
             x_ref, agg_ref, out_ref):
        hin = (1.0 + eps_ref[0, 0]) * x_ref[...] + (agg_ref[0] + agg_ref[1])
        o = _mlp_block(hin, W1_ref, b1_ref, g_ref, be_ref, W2_ref, b2_ref)
        h1 = jnp.maximum(o, 0.0)
        out_ref[0] = h1[:, :128]
        out_ref[1] = h1[:, 128:]

    return pl.pallas_call(
        body,
        grid=(N // BR,),
        in_specs=_wspecs(128) + [
            pl.BlockSpec((BR, 128), lambda i: (i, 0)),
            pl.BlockSpec((2, BR, 128), lambda i: (0, i, 0)),
        ],
        out_specs=pl.BlockSpec((2, BR, 128), lambda i: (0, i, 0)),
        out_shape=jax.ShapeDtypeStruct((2, N, 128), jnp.float32),
    )(eps.reshape(1, 1), W1, b1.reshape(1, 256), g.reshape(1, 256),
      be.reshape(1, 256), W2, b2.reshape(1, 256), x, agg)


def _tc_layer_mid(hh, agg, eps, W1, b1, g, be, W2, b2):
    def body(eps_ref, W1_ref, b1_ref, g_ref, be_ref, W2_ref, b2_ref,
             hh_ref, agg_ref, out_ref):
        h = jnp.concatenate([hh_ref[0], hh_ref[1]], axis=-1)
        hin = (1.0 + eps_ref[0, 0]) * h + agg_ref[...]
        o = _mlp_block(hin, W1_ref, b1_ref, g_ref, be_ref, W2_ref, b2_ref)
        h2 = h + jnp.maximum(o, 0.0)
        out_ref[0] = h2[:, :128]
        out_ref[1] = h2[:, 128:]

    return pl.pallas_call(
        body,
        grid=(N // BR,),
        in_specs=_wspecs(256) + [
            pl.BlockSpec((2, BR, 128), lambda i: (0, i, 0)),
            pl.BlockSpec((BR, 256), lambda i: (i, 0)),
        ],
        out_specs=pl.BlockSpec((2, BR, 128), lambda i: (0, i, 0)),
        out_shape=jax.ShapeDtypeStruct((2, N, 128), jnp.float32),
    )(eps.reshape(1, 1), W1, b1.reshape(1, 256), g.reshape(1, 256),
      be.reshape(1, 256), W2, b2.reshape(1, 256), hh, agg)


def _tc_layer_last(hh, agg, eps, W1, b1, g, be, W2, b2, Wo_pad, bo_pad):
    def body(eps_ref, W1_ref, b1_ref, g_ref, be_ref, W2_ref, b2_ref,
             Wo_ref, bo_ref, hh_ref, agg_ref, out_ref):
        h = jnp.concatenate([hh_ref[0], hh_ref[1]], axis=-1)
        hin = (1.0 + eps_ref[0, 0]) * h + agg_ref[...]
        o = _mlp_block(hin, W1_ref, b1_ref, g_ref, be_ref, W2_ref, b2_ref)
        h3 = h + jnp.maximum(o, 0.0)
        out_ref[...] = (jnp.dot(h3, Wo_ref[...], preferred_element_type=jnp.float32)
                        + bo_ref[...])

    return pl.pallas_call(
        body,
        grid=(N // BR,),
        in_specs=_wspecs(256) + [
            pl.BlockSpec((256, 128), lambda i: (0, 0)),
            pl.BlockSpec((1, 128), lambda i: (0, 0)),
            pl.BlockSpec((2, BR, 128), lambda i: (0, i, 0)),
            pl.BlockSpec((BR, 256), lambda i: (i, 0)),
        ],
        out_specs=pl.BlockSpec((BR, 128), lambda i: (i, 0)),
        out_shape=jax.ShapeDtypeStruct((N, 128), jnp.float32),
    )(eps.reshape(1, 1), W1, b1.reshape(1, 256), g.reshape(1, 256),
      be.reshape(1, 256), W2, b2.reshape(1, 256), Wo_pad, bo_pad, hh, agg)


def kernel(x, edge_index,
           W1_0, b1_0, g_0, be_0, W2_0, b2_0, eps_0,
           W1_1, b1_1, g_1, be_1, W2_1, b2_1, eps_1,
           W1_2, b1_2, g_2, be_2, W2_2, b2_2, eps_2,
           W_out, b_out):
    src = edge_index[0].astype(jnp.int32)
    dst = edge_index[1].astype(jnp.int32)
    srcs = jnp.stack([src, src + N]).reshape(2, NB_F, 2, 5, SUB_F)
    dst2 = dst.reshape(NB_F, 2, 5, SUB_F)
    src0 = src.reshape(NB_F, 2, 5, SUB_F)
    dst0 = dst.reshape(NB_F, 2, 5, SUB_F)
    z128 = jnp.zeros((N, 128), jnp.float32)

    agg0 = _SC_AGG_L0(x, src0, dst0, z128)
    h1h = _tc_layer0(x, agg0, eps_0, W1_0, b1_0, g_0, be_0, W2_0, b2_0)
    agg1 = _SC_AGG128(h1h.reshape(2 * N, 128), srcs, dst2, z128)
    h2h = _tc_layer_mid(h1h, agg1, eps_1, W1_1, b1_1, g_1, be_1, W2_1, b2_1)
    agg2 = _SC_AGG128(h2h.reshape(2 * N, 128), srcs, dst2, z128)
    Wo_pad = jnp.pad(W_out, ((0, 0), (0, 126)))
    bo_pad = jnp.pad(b_out, (0, 126)).reshape(1, 128)
    outp = _tc_layer_last(h2h, agg2, eps_2, W1_2, b1_2, g_2, be_2, W2_2, b2_2,
                          Wo_pad, bo_pad)
    return outp[:, :2]


# speedup vs baseline: 1.0978x; 1.0433x over previous
import functools

import jax
import jax.numpy as jnp
from jax import lax
from jax.experimental import pallas as pl
from jax.experimental.pallas import tpu as pltpu
from jax.experimental.pallas import tpu_sc as plsc

N = 10000
E = 320000
SUB = 100
NB = 320
NSLOT = 3
RPT = 632
RPT_LAST = N - 15 * RPT
SPLIT = 15 * RPT


def _make_sc_agg(split_edges):
    mesh = plsc.VectorSubcoreMesh(core_axis_name="c", subcore_axis_name="s")
    if split_edges:
        out_type = jax.ShapeDtypeStruct((2, N, 128), jnp.float32)
        nbody = NB // 32
    else:
        out_type = jax.ShapeDtypeStruct((N, 256), jnp.float32)
        nbody = NB // 16

    def body(h_hbm, src_hbm, dst_hbm, zeros_hbm, out_hbm,
             srcA, srcB, dstA, dstB, rows_v, agg_sh,
             sem_g, sem_s, sem_iA, sem_iB):
        c = lax.axis_index("c")
        s = lax.axis_index("s")
        if split_edges:
            m0 = (s * 2 + c) * nbody
            def src_slice(m, half):
                return src_hbm.at[m, half]
        else:
            m0 = s * nbody
            def src_slice(m, half):
                return src_hbm.at[c, m, half]

        def init(r0, nrows):
            pltpu.sync_copy(zeros_hbm.at[pl.ds(r0, nrows)],
                            agg_sh.at[pl.ds(r0, nrows)])

        def copy_out(r0, nrows):
            if split_edges:
                pltpu.sync_copy(agg_sh.at[pl.ds(r0, nrows)],
                                out_hbm.at[c, pl.ds(r0, nrows)])
            else:
                pltpu.sync_copy(agg_sh.at[pl.ds(r0, nrows)],
                                out_hbm.at[pl.ds(r0, nrows),
                                           pl.ds(c * 128, 128)])

        @pl.when(s < 15)
        def _():
            init(s * RPT, RPT)
        @pl.when(s == 15)
        def _():
            init(SPLIT, RPT_LAST)
        plsc.subcore_barrier()

        pltpu.sync_copy(src_slice(m0, 0), srcA)
        pltpu.sync_copy(dst_hbm.at[m0, 0], dstA)

        def drain_scatter(b):
            pltpu.make_async_copy(rows_v.at[b], agg_sh.at[dstA.at[0]],
                                  sem_s.at[b]).wait()

        def drain_idx(sem, src_ref, dst_ref):
            pltpu.make_async_copy(src_slice(m0, 0), src_ref, sem).wait()
            pltpu.make_async_copy(dst_hbm.at[m0, 0], dst_ref, sem).wait()

        @pl.loop(0, nbody)
        def _body(t):
            m = m0 + t
            @pl.when(t > 0)
            def _():
                drain_scatter(1)
                drain_scatter(2)
                drain_scatter(0)
                drain_idx(sem_iA, srcA, dstA)

            gather_descs = {}

            def fire_gather(j):
                b = j % NSLOT
                si = (srcA if j < 5 else srcB).at[j % 5]
                gather_descs[j] = pltpu.async_copy(
                    h_hbm.at[si], rows_v.at[b], sem_g.at[b])

            def fire_scatter(j):
                b = j % NSLOT
                di = (dstA if j < 5 else dstB).at[j % 5]
                gather_descs[j].wait()
                pltpu.async_copy(rows_v.at[b], agg_sh.at[di],
                                 sem_s.at[b], add=True)

            for j in range(10):
                if 3 <= j:
                    drain_scatter(j % NSLOT)
                fire_gather(j)
                if j == 2:
                    pltpu.async_copy(src_slice(m, 1), srcB, sem_iB)
                    pltpu.async_copy(dst_hbm.at[m, 1], dstB, sem_iB)
                if j == 5:
                    pltpu.make_async_copy(src_slice(m, 1), srcB, sem_iB).wait()
                    pltpu.make_async_copy(dst_hbm.at[m, 1], dstB, sem_iB).wait()
                if j == 8:
                    mn = jnp.minimum(m + 1, NB - 1)
                    pltpu.async_copy(src_slice(mn, 0), srcA, sem_iA)
                    pltpu.async_copy(dst_hbm.at[mn, 0], dstA, sem_iA)
                if j >= 1:
                    fire_scatter(j - 1)
            fire_scatter(9)

        drain_scatter(1)
        drain_scatter(2)
        drain_scatter(0)
        drain_idx(sem_iA, srcA, dstA)

        plsc.subcore_barrier()

        @pl.when(s < 15)
        def _():
            copy_out(s * RPT, RPT)
        @pl.when(s == 15)
        def _():
            copy_out(SPLIT, RPT_LAST)

    return pl.kernel(
        body,
        out_type=out_type,
        mesh=mesh,
        scratch_types=[
            pltpu.VMEM((5, SUB), jnp.int32),
            pltpu.VMEM((5, SUB), jnp.int32),
            pltpu.VMEM((5, SUB), jnp.int32),
            pltpu.VMEM((5, SUB), jnp.int32),
            pltpu.VMEM((NSLOT, SUB, 128), jnp.float32),
            pltpu.VMEM_SHARED((N, 128), jnp.float32),
            pltpu.SemaphoreType.DMA((NSLOT,)),
            pltpu.SemaphoreType.DMA((NSLOT,)),
            pltpu.SemaphoreType.DMA,
            pltpu.SemaphoreType.DMA,
        ],
    )


_SC_AGG128 = _make_sc_agg(split_edges=False)
_SC_AGG_L0 = _make_sc_agg(split_edges=True)

BR = 2000


def _mlp_block(hin, W1_ref, b1_ref, g_ref, be_ref, W2_ref, b2_ref):
    z = jnp.dot(hin, W1_ref[...], preferred_element_type=jnp.float32) + b1_ref[...]
    mu = jnp.mean(z, axis=-1, keepdims=True)
    zc = z - mu
    var = jnp.mean(zc * zc, axis=-1, keepdims=True)
    zn = zc * lax.rsqrt(var + 1e-5) * g_ref[...] + be_ref[...]
    za = jnp.maximum(zn, 0.0)
    return jnp.dot(za, W2_ref[...], preferred_element_type=jnp.float32) + b2_ref[...]


def _wspecs(din):
    return [
        pl.BlockSpec((1, 1), lambda i: (0, 0)),
        pl.BlockSpec((din, 256), lambda i: (0, 0)),
        pl.BlockSpec((1, 256), lambda i: (0, 0)),
        pl.BlockSpec((1, 256), lambda i: (0, 0)),
        pl.BlockSpec((1, 256), lambda i: (0, 0)),
        pl.BlockSpec((256, 256), lambda i: (0, 0)),
        pl.BlockSpec((1, 256), lambda i: (0, 0)),
    ]


def _tc_layer0(x, agg, eps, W1, b1, g, be, W2, b2):
    def body(eps_ref, W1_ref, b1_ref, g_ref, be_ref, W2_ref, b2_ref,
             x_ref, agg_ref, out_ref):
        hin = (1.0 + eps_ref[0, 0]) * x_ref[...] + (agg_ref[0] + agg_ref[1])
        o = _mlp_block(hin, W1_ref, b1_ref, g_ref, be_ref, W2_ref, b2_ref)
        h1 = jnp.maximum(o, 0.0)
        out_ref[0] = h1[:, :128]
        out_ref[1] = h1[:, 128:]

    return pl.pallas_call(
        body,
        grid=(N // BR,),
        in_specs=_wspecs(128) + [
            pl.BlockSpec((BR, 128), lambda i: (i, 0)),
            pl.BlockSpec((2, BR, 128), lambda i: (0, i, 0)),
        ],
        out_specs=pl.BlockSpec((2, BR, 128), lambda i: (0, i, 0)),
        out_shape=jax.ShapeDtypeStruct((2, N, 128), jnp.float32),
    )(eps.reshape(1, 1), W1, b1.reshape(1, 256), g.reshape(1, 256),
      be.reshape(1, 256), W2, b2.reshape(1, 256), x, agg)


def _tc_layer_mid(hh, agg, eps, W1, b1, g, be, W2, b2):
    def body(eps_ref, W1_ref, b1_ref, g_ref, be_ref, W2_ref, b2_ref,
             hh_ref, agg_ref, out_ref):
        h = jnp.concatenate([hh_ref[0], hh_ref[1]], axis=-1)
        hin = (1.0 + eps_ref[0, 0]) * h + agg_ref[...]
        o = _mlp_block(hin, W1_ref, b1_ref, g_ref, be_ref, W2_ref, b2_ref)
        h2 = h + jnp.maximum(o, 0.0)
        out_ref[0] = h2[:, :128]
        out_ref[1] = h2[:, 128:]

    return pl.pallas_call(
        body,
        grid=(N // BR,),
        in_specs=_wspecs(256) + [
            pl.BlockSpec((2, BR, 128), lambda i: (0, i, 0)),
            pl.BlockSpec((BR, 256), lambda i: (i, 0)),
        ],
        out_specs=pl.BlockSpec((2, BR, 128), lambda i: (0, i, 0)),
        out_shape=jax.ShapeDtypeStruct((2, N, 128), jnp.float32),
    )(eps.reshape(1, 1), W1, b1.reshape(1, 256), g.reshape(1, 256),
      be.reshape(1, 256), W2, b2.reshape(1, 256), hh, agg)


def _tc_layer_last(hh, agg, eps, W1, b1, g, be, W2, b2, Wo_pad, bo_pad):
    def body(eps_ref, W1_ref, b1_ref, g_ref, be_ref, W2_ref, b2_ref,
             Wo_ref, bo_ref, hh_ref, agg_ref, out_ref):
        h = jnp.concatenate([hh_ref[0], hh_ref[1]], axis=-1)
        hin = (1.0 + eps_ref[0, 0]) * h + agg_ref[...]
        o = _mlp_block(hin, W1_ref, b1_ref, g_ref, be_ref, W2_ref, b2_ref)
        h3 = h + jnp.maximum(o, 0.0)
        out_ref[...] = (jnp.dot(h3, Wo_ref[...], preferred_element_type=jnp.float32)
                        + bo_ref[...])

    return pl.pallas_call(
        body,
        grid=(N // BR,),
        in_specs=_wspecs(256) + [
            pl.BlockSpec((256, 128), lambda i: (0, 0)),
            pl.BlockSpec((1, 128), lambda i: (0, 0)),
            pl.BlockSpec((2, BR, 128), lambda i: (0, i, 0)),
            pl.BlockSpec((BR, 256), lambda i: (i, 0)),
        ],
        out_specs=pl.BlockSpec((BR, 128), lambda i: (i, 0)),
        out_shape=jax.ShapeDtypeStruct((N, 128), jnp.float32),
    )(eps.reshape(1, 1), W1, b1.reshape(1, 256), g.reshape(1, 256),
      be.reshape(1, 256), W2, b2.reshape(1, 256), Wo_pad, bo_pad, hh, agg)


def kernel(x, edge_index,
           W1_0, b1_0, g_0, be_0, W2_0, b2_0, eps_0,
           W1_1, b1_1, g_1, be_1, W2_1, b2_1, eps_1,
           W1_2, b1_2, g_2, be_2, W2_2, b2_2, eps_2,
           W_out, b_out):
    src = edge_index[0].astype(jnp.int32)
    dst = edge_index[1].astype(jnp.int32)
    srcs = jnp.stack([src, src + N]).reshape(2, NB, 2, 5, SUB)
    src0 = src.reshape(NB, 2, 5, SUB)
    dst2 = dst.reshape(NB, 2, 5, SUB)
    z128 = jnp.zeros((N, 128), jnp.float32)

    agg0 = _SC_AGG_L0(x, src0, dst2, z128)
    h1h = _tc_layer0(x, agg0, eps_0, W1_0, b1_0, g_0, be_0, W2_0, b2_0)
    agg1 = _SC_AGG128(h1h.reshape(2 * N, 128), srcs, dst2, z128)
    h2h = _tc_layer_mid(h1h, agg1, eps_1, W1_1, b1_1, g_1, be_1, W2_1, b2_1)
    agg2 = _SC_AGG128(h2h.reshape(2 * N, 128), srcs, dst2, z128)
    Wo_pad = jnp.pad(W_out, ((0, 0), (0, 126)))
    bo_pad = jnp.pad(b_out, (0, 126)).reshape(1, 128)
    outp = _tc_layer_last(h2h, agg2, eps_2, W1_2, b1_2, g_2, be_2, W2_2, b2_2,
                          Wo_pad, bo_pad)
    return outp[:, :2]
